# baseline (speedup 1.0000x reference)
import jax
import jax.numpy as jnp
from jax.experimental import pallas as pl
from jax.experimental.pallas import tpu as pltpu

_B = 128
_NT = 512
_CD = 512
_L = 16
_ROWS = _B * _L
_H = 2048
_OUT = 3 * 2048
_EPS = 1e-5

_HIGH = jax.lax.Precision.HIGHEST
_NT_DIMS = (((1,), (1,)), ((), ()))

_CP = pltpu.CompilerParams(vmem_limit_bytes=100 * 1024 * 1024)


def _vq_body(flat_ref, cb_ref, fn_ref, cn_ref, idx_ref, quant_ref, loss_ref):
    flat = flat_ref[...]
    cb = cb_ref[...]
    s = jax.lax.dot_general(flat, cb, _NT_DIMS)
    dist = fn_ref[...] + cn_ref[...] - 2.0 * s
    minval = jnp.min(dist, axis=1, keepdims=True)
    jj = jax.lax.broadcasted_iota(jnp.int32, dist.shape, 1)
    idx = jnp.min(jnp.where(dist == minval, jj, _NT), axis=1)
    onehot = (jj == idx[:, None]).astype(jnp.float32)
    quant = jax.lax.dot_general(onehot, cb, (((1,), (0,)), ((), ())),
                                precision=_HIGH)
    diff = quant - flat
    loss_ref[...] = (1.25 * jnp.mean(diff * diff)).reshape(1, 1)
    idx_ref[...] = idx.reshape(_B, _L)
    quant_ref[...] = flat + diff


def _bn_relu(h, g, be):
    m = jnp.mean(h, axis=0, keepdims=True)
    d = h - m
    v = jnp.mean(d * d, axis=0, keepdims=True)
    return jnp.maximum(g * d / jnp.sqrt(v + _EPS) + be, 0.0)


def _conv_bn_body(x_ref, w_ref, b_ref, g_ref, be_ref, o_ref):
    h = jax.lax.dot_general(x_ref[...], w_ref[...], _NT_DIMS) + b_ref[...]
    o_ref[...] = _bn_relu(h, g_ref[...], be_ref[...])


def _conv_res_body(x_ref, w_ref, b_ref, res_ref, o_ref):
    h = jax.lax.dot_general(x_ref[...], w_ref[...], _NT_DIMS) + b_ref[...]
    o_ref[...] = res_ref[...] + h


def _end_a_body(x_ref, w_ref, b_ref, g_ref, be_ref, o_ref):
    mx = jnp.max(x_ref[...].reshape(_B, _L, _CD), axis=1)
    h = jnp.maximum(mx, 0.0)
    h = jax.lax.dot_general(h, w_ref[...], _NT_DIMS) + b_ref[...]
    o_ref[...] = _bn_relu(h, g_ref[...], be_ref[...])


def _end_b_body(h_ref, w_ref, b_ref, o_ref):
    o_ref[...] = (jax.lax.dot_general(h_ref[...], w_ref[...], _NT_DIMS)
                  + b_ref[...])


def _tiled_conv_bn(x, w, b, g, be, n_out, tile):
    cin = x.shape[1]
    grid = n_out // tile
    return pl.pallas_call(
        _conv_bn_body,
        grid=(grid,),
        in_specs=[
            pl.BlockSpec((_ROWS, cin), lambda i: (0, 0)),
            pl.BlockSpec((tile, cin), lambda i: (i, 0)),
            pl.BlockSpec((1, tile), lambda i: (0, i)),
            pl.BlockSpec((1, tile), lambda i: (0, i)),
            pl.BlockSpec((1, tile), lambda i: (0, i)),
        ],
        out_specs=pl.BlockSpec((_ROWS, tile), lambda i: (0, i)),
        out_shape=jax.ShapeDtypeStruct((_ROWS, n_out), jnp.float32),
        compiler_params=_CP,
    )(x, w, b, g, be)


def _tiled_conv_res(x, w, b, res, n_out, tile):
    cin = x.shape[1]
    grid = n_out // tile
    return pl.pallas_call(
        _conv_res_body,
        grid=(grid,),
        in_specs=[
            pl.BlockSpec((_ROWS, cin), lambda i: (0, 0)),
            pl.BlockSpec((tile, cin), lambda i: (i, 0)),
            pl.BlockSpec((1, tile), lambda i: (0, i)),
            pl.BlockSpec((_ROWS, tile), lambda i: (0, i)),
        ],
        out_specs=pl.BlockSpec((_ROWS, tile), lambda i: (0, i)),
        out_shape=jax.ShapeDtypeStruct((_ROWS, n_out), jnp.float32),
        compiler_params=_CP,
    )(x, w, b, res)


def _row(a):
    return a[None, :]


def kernel(z, codebook, params):
    flat4 = jnp.transpose(z[:, :, :, None], (0, 2, 3, 1)).reshape(-1, _NT)
    fn = jnp.sum(flat4 ** 2, axis=1, keepdims=True)
    cn = jnp.sum(codebook ** 2, axis=1)[None, :]
    idx, quant, loss = pl.pallas_call(
        _vq_body,
        out_shape=(
            jax.ShapeDtypeStruct((_B, _L), jnp.int32),
            jax.ShapeDtypeStruct((_ROWS, _CD), jnp.float32),
            jax.ShapeDtypeStruct((1, 1), jnp.float32),
        ),
        compiler_params=_CP,
    )(flat4, codebook, fn, cn)

    x = quant
    for name in ('f1', 'f2', 'f3'):
        p = params[name]
        h = _tiled_conv_bn(x, p['W1'], _row(p['b1']), _row(p['g1']),
                           _row(p['be1']), _H, 512)
        h = _tiled_conv_bn(h, p['W2'], _row(p['b2']), _row(p['g2']),
                           _row(p['be2']), _H, 512)
        x = _tiled_conv_res(h, p['W3'], _row(p['b3']), x, _CD, 256)

    e = params['end']
    h = pl.pallas_call(
        _end_a_body,
        out_shape=jax.ShapeDtypeStruct((_B, 1024), jnp.float32),
        compiler_params=_CP,
    )(x, e['W1'], _row(e['b1']), _row(e['g1']), _row(e['be1']))

    grid = _OUT // 512
    out = pl.pallas_call(
        _end_b_body,
        grid=(grid,),
        in_specs=[
            pl.BlockSpec((_B, 1024), lambda i: (0, 0)),
            pl.BlockSpec((512, 1024), lambda i: (i, 0)),
            pl.BlockSpec((1, 512), lambda i: (0, i)),
        ],
        out_specs=pl.BlockSpec((_B, 512), lambda i: (0, i)),
        out_shape=jax.ShapeDtypeStruct((_B, _OUT), jnp.float32),
        compiler_params=_CP,
    )(h, e['W2'], _row(e['b2']))

    return (loss[0, 0], out[:, :, None], idx)


# speedup vs baseline: 1.3029x; 1.3029x over previous
import jax
import jax.numpy as jnp
from jax.experimental import pallas as pl
from jax.experimental.pallas import tpu as pltpu

_B = 128
_NT = 512
_CD = 512
_L = 16
_ROWS = _B * _L
_H = 2048
_OUT = 3 * 2048
_EPS = 1e-5

_HIGH = jax.lax.Precision.HIGHEST
_NT_DIMS = (((1,), (1,)), ((), ()))

_CP = pltpu.CompilerParams(vmem_limit_bytes=100 * 1024 * 1024)
_VMEM_SPEC = pl.BlockSpec(memory_space=pltpu.MemorySpace.VMEM)
_HBM_SPEC = pl.BlockSpec(memory_space=pltpu.MemorySpace.HBM)


def _vq_body(flat_ref, cb_ref, fn_ref, cn_ref, idx_ref, quant_ref, loss_ref):
    flat = flat_ref[...]
    cb = cb_ref[...]
    s = jax.lax.dot_general(flat, cb, _NT_DIMS)
    dist = fn_ref[...] + cn_ref[...] - 2.0 * s
    minval = jnp.min(dist, axis=1, keepdims=True)
    jj = jax.lax.broadcasted_iota(jnp.int32, dist.shape, 1)
    idx = jnp.min(jnp.where(dist == minval, jj, _NT), axis=1)
    onehot = (jj == idx[:, None]).astype(jnp.float32)
    quant = jax.lax.dot_general(onehot, cb, (((1,), (0,)), ((), ())),
                                precision=_HIGH)
    diff = quant - flat
    loss_ref[...] = (1.25 * jnp.mean(diff * diff)).reshape(1, 1)
    idx_ref[...] = idx.reshape(_B, _L)
    quant_ref[...] = flat + diff


def _bn_relu(h, g, be):
    m = jnp.mean(h, axis=0, keepdims=True)
    d = h - m
    v = jnp.mean(d * d, axis=0, keepdims=True)
    return jnp.maximum(g * d / jnp.sqrt(v + _EPS) + be, 0.0)


_CH = 512
_NCK = _H // _CH


def _fold_body(x_ref, w1_hbm, b1_ref, g1_ref, be1_ref,
               w2_hbm, b2_ref, g2_ref, be2_ref, w3_hbm, b3_ref,
               y_ref, w1_ref, w2_ref, w3_ref, sem):
    c1 = pltpu.make_async_copy(w1_hbm, w1_ref, sem.at[0])
    c3 = pltpu.make_async_copy(w3_hbm, w3_ref, sem.at[1])
    c2 = [pltpu.make_async_copy(w2_hbm.at[pl.ds(k * _CH, _CH), :],
                                w2_ref.at[k % 2], sem.at[2 + (k % 2)])
          for k in range(_NCK)]
    c1.start()
    c2[0].start()
    c2[1].start()
    c3.start()
    x = x_ref[...]
    c1.wait()
    h = jax.lax.dot_general(x, w1_ref[...], _NT_DIMS) + b1_ref[...]
    h = _bn_relu(h, g1_ref[...], be1_ref[...])
    chunks = []
    for k in range(_NCK):
        c2[k].wait()
        wchunk = w2_ref[k % 2]
        chunks.append(jax.lax.dot_general(h, wchunk, _NT_DIMS))
        if k + 2 < _NCK:
            c2[k + 2].start()
    h = jnp.concatenate(chunks, axis=1) + b2_ref[...]
    h = _bn_relu(h, g2_ref[...], be2_ref[...])
    c3.wait()
    h = jax.lax.dot_general(h, w3_ref[...], _NT_DIMS) + b3_ref[...]
    y_ref[...] = x + h


def _end_body(x_ref, w1_ref, b1_ref, g1_ref, be1_ref, w2_hbm, b2_ref,
              out_ref, w2_ref, sem):
    c2 = pltpu.make_async_copy(w2_hbm, w2_ref, sem.at[0])
    c2.start()
    mx = jnp.max(x_ref[...].reshape(_B, _L, _CD), axis=1)
    h = jnp.maximum(mx, 0.0)
    h = jax.lax.dot_general(h, w1_ref[...], _NT_DIMS) + b1_ref[...]
    h = _bn_relu(h, g1_ref[...], be1_ref[...])
    c2.wait()
    out_ref[...] = (jax.lax.dot_general(h, w2_ref[...], _NT_DIMS)
                    + b2_ref[...])


def _row(a):
    return a[None, :]


def kernel(z, codebook, params):
    flat4 = jnp.transpose(z[:, :, :, None], (0, 2, 3, 1)).reshape(-1, _NT)
    fn = jnp.sum(flat4 ** 2, axis=1, keepdims=True)
    cn = jnp.sum(codebook ** 2, axis=1)[None, :]
    idx, quant, loss = pl.pallas_call(
        _vq_body,
        out_shape=(
            jax.ShapeDtypeStruct((_B, _L), jnp.int32),
            jax.ShapeDtypeStruct((_ROWS, _CD), jnp.float32),
            jax.ShapeDtypeStruct((1, 1), jnp.float32),
        ),
        compiler_params=_CP,
    )(flat4, codebook, fn, cn)

    fold = pl.pallas_call(
        _fold_body,
        in_specs=[_VMEM_SPEC, _HBM_SPEC, _VMEM_SPEC, _VMEM_SPEC, _VMEM_SPEC,
                  _HBM_SPEC, _VMEM_SPEC, _VMEM_SPEC, _VMEM_SPEC,
                  _HBM_SPEC, _VMEM_SPEC],
        out_shape=jax.ShapeDtypeStruct((_ROWS, _CD), jnp.float32),
        scratch_shapes=[
            pltpu.VMEM((_H, _CD), jnp.float32),
            pltpu.VMEM((2, _CH, _H), jnp.float32),
            pltpu.VMEM((_CD, _H), jnp.float32),
            pltpu.SemaphoreType.DMA((4,)),
        ],
        compiler_params=_CP,
    )
    x = quant
    for name in ('f1', 'f2', 'f3'):
        p = params[name]
        x = fold(x, p['W1'], _row(p['b1']), _row(p['g1']), _row(p['be1']),
                 p['W2'], _row(p['b2']), _row(p['g2']), _row(p['be2']),
                 p['W3'], _row(p['b3']))

    e = params['end']
    out = pl.pallas_call(
        _end_body,
        in_specs=[_VMEM_SPEC, _VMEM_SPEC, _VMEM_SPEC, _VMEM_SPEC, _VMEM_SPEC,
                  _HBM_SPEC, _VMEM_SPEC],
        out_shape=jax.ShapeDtypeStruct((_B, _OUT), jnp.float32),
        scratch_shapes=[
            pltpu.VMEM((_OUT, 1024), jnp.float32),
            pltpu.SemaphoreType.DMA((1,)),
        ],
        compiler_params=_CP,
    )(x, e['W1'], _row(e['b1']), _row(e['g1']), _row(e['be1']),
      e['W2'], _row(e['b2']))

    return (loss[0, 0], out[:, :, None], idx)


# speedup vs baseline: 1.3605x; 1.0442x over previous
import jax
import jax.numpy as jnp
from jax.experimental import pallas as pl
from jax.experimental.pallas import tpu as pltpu

_B = 128
_NT = 512
_CD = 512
_L = 16
_ROWS = _B * _L
_EPS = 1e-5

_HIGH = jax.lax.Precision.HIGHEST
_NT_DIMS = (((1,), (1,)), ((), ()))


def _vq_body(flat_ref, cb_ref, fn_ref, cn_ref, idx_ref, quant_ref, loss_ref):
    flat = flat_ref[...]
    cb = cb_ref[...]
    s = jax.lax.dot_general(flat, cb, _NT_DIMS)
    dist = fn_ref[...] + cn_ref[...] - 2.0 * s
    minval = jnp.min(dist, axis=1, keepdims=True)
    jj = jax.lax.broadcasted_iota(jnp.int32, dist.shape, 1)
    idx = jnp.min(jnp.where(dist == minval, jj, _NT), axis=1)
    onehot = (jj == idx[:, None]).astype(jnp.float32)
    quant = jax.lax.dot_general(onehot, cb, (((1,), (0,)), ((), ())),
                                precision=_HIGH)
    diff = quant - flat
    loss_ref[...] = (1.25 * jnp.mean(diff * diff)).reshape(1, 1)
    idx_ref[...] = idx.reshape(_B, _L)
    quant_ref[...] = flat + diff


def _bn_relu(h, g, be):
    m = jnp.mean(h, axis=0, keepdims=True)
    d = h - m
    v = jnp.mean(d * d, axis=0, keepdims=True)
    return jnp.maximum(g[None, :] * d / jnp.sqrt(v + _EPS) + be[None, :], 0.0)


def _fold_body(x_ref, w1_ref, b1_ref, g1_ref, be1_ref,
               w2_ref, b2_ref, g2_ref, be2_ref,
               w3_ref, b3_ref, y_ref):
    x = x_ref[...]
    h = jax.lax.dot_general(x, w1_ref[...], _NT_DIMS)
    h = _bn_relu(h + b1_ref[...][None, :], g1_ref[...], be1_ref[...])
    h = jax.lax.dot_general(h, w2_ref[...], _NT_DIMS)
    h = _bn_relu(h + b2_ref[...][None, :], g2_ref[...], be2_ref[...])
    h = jax.lax.dot_general(h, w3_ref[...], _NT_DIMS)
    y_ref[...] = x + (h + b3_ref[...][None, :])


def _end_body(x_ref, w1_ref, b1_ref, g1_ref, be1_ref, w2_ref, b2_ref,
              out_ref):
    x = x_ref[...]
    mx = jnp.max(x.reshape(_B, _L, _CD), axis=1)
    h = jnp.maximum(mx, 0.0)
    h = jax.lax.dot_general(h, w1_ref[...], _NT_DIMS)
    h = _bn_relu(h + b1_ref[...][None, :], g1_ref[...], be1_ref[...])
    out = jax.lax.dot_general(h, w2_ref[...], _NT_DIMS)
    out_ref[...] = out + b2_ref[...][None, :]


_CP = pltpu.CompilerParams(vmem_limit_bytes=100 * 1024 * 1024)


def kernel(z, codebook, params):
    flat4 = jnp.transpose(z[:, :, :, None], (0, 2, 3, 1)).reshape(-1, _NT)
    fn = jnp.sum(flat4 ** 2, axis=1, keepdims=True)
    cn = jnp.sum(codebook ** 2, axis=1)[None, :]
    idx, quant, loss = pl.pallas_call(
        _vq_body,
        out_shape=(
            jax.ShapeDtypeStruct((_B, _L), jnp.int32),
            jax.ShapeDtypeStruct((_ROWS, _CD), jnp.float32),
            jax.ShapeDtypeStruct((1, 1), jnp.float32),
        ),
        compiler_params=_CP,
    )(flat4, codebook, fn, cn)

    fold = pl.pallas_call(
        _fold_body,
        out_shape=jax.ShapeDtypeStruct((_ROWS, _CD), jnp.float32),
        compiler_params=_CP,
    )
    x = quant
    for name in ('f1', 'f2', 'f3'):
        p = params[name]
        x = fold(x, p['W1'], p['b1'], p['g1'], p['be1'],
                 p['W2'], p['b2'], p['g2'], p['be2'], p['W3'], p['b3'])

    e = params['end']
    out = pl.pallas_call(
        _end_body,
        out_shape=jax.ShapeDtypeStruct((_B, 3 * 2048), jnp.float32),
        compiler_params=_CP,
    )(x, e['W1'], e['b1'], e['g1'], e['be1'], e['W2'], e['b2'])

    return (loss[0, 0], out[:, :, None], idx)
